# Initial kernel scaffold; baseline (speedup 1.0000x reference)
import jax
import jax.numpy as jnp
from jax.experimental import pallas as pl


def kernel(features, b_idx, d_idx, h_idx, w_idx):
    raise NotImplementedError("write your pallas kernel here")


# speedup vs baseline: 1.0989x; 1.0989x over previous
import functools

import jax
import jax.numpy as jnp
from jax import lax
from jax.experimental import pallas as pl
from jax.experimental.pallas import tpu as pltpu
from jax.experimental.pallas import tpu_sc as plsc

N_BATCH, C, D, H, W = 4, 128, 5, 75, 75
HW = H * W
HWP = 5632
PLANES = N_BATCH * D
RINTER = PLANES * HWP
NNZ = 40000
VPAD = 40960
CHUNK = 256
CHUNKS_PER_TILE = 5
NC, NS, L = 2, 16, 16
NW = NC * NS
ZSLICE = (NC * RINTER) // NW
ZBUF = 704


def _sc_body(feat, b_hbm, d_hbm, h_hbm, w_hbm, inter, valid,
             bbuf, dbuf, hbuf, wbuf, fbuf, destb, flagb, zbuf, obuf):
    c = lax.axis_index("c")
    s = lax.axis_index("s")
    wid = s * NC + c

    zv = jnp.zeros((L,), jnp.float32)
    for k in range(ZBUF // L):
        zbuf[pl.ds(k * L, L)] = zv
    ov = jnp.full((L,), 1.0, jnp.float32)
    for k in range(CHUNK // (2 * L)):
        obuf[pl.ds(k * L, L)] = ov

    zbase = c * RINTER + s * ZSLICE
    for t in range(ZSLICE // ZBUF):
        pltpu.sync_copy(zbuf, valid.at[pl.ds(zbase + t * ZBUF, ZBUF)])

    plsc.subcore_barrier()

    trash = (wid % PLANES) * HWP + HW + (wid // PLANES)
    for j in range(CHUNKS_PER_TILE):
        cid = wid * CHUNKS_PER_TILE + j
        vb = cid * CHUNK
        pltpu.sync_copy(b_hbm.at[pl.ds(vb, CHUNK)], bbuf)
        pltpu.sync_copy(d_hbm.at[pl.ds(vb, CHUNK)], dbuf)
        pltpu.sync_copy(h_hbm.at[pl.ds(vb, CHUNK)], hbuf)
        pltpu.sync_copy(w_hbm.at[pl.ds(vb, CHUNK)], wbuf)
        for v in range(CHUNK // L):
            off = v * L
            bb = bbuf[pl.ds(off, L)]
            dd = dbuf[pl.ds(off, L)]
            hh = hbuf[pl.ds(off, L)]
            ww = wbuf[pl.ds(off, L)]
            vid = vb + off + lax.iota(jnp.int32, L)
            r = (bb * D + dd) * HWP + hh * W + ww
            dest = jnp.where(vid < NNZ, r, trash)
            hv, lo = off // (CHUNK // 2), off % (CHUNK // 2)
            destb[hv, pl.ds(lo, L)] = dest
            flagb[hv, pl.ds(lo, L)] = dest + c * RINTER

        @pl.when(vb + CHUNK <= NNZ)
        def _full():
            pltpu.sync_copy(feat.at[pl.ds(vb, CHUNK)], fbuf)

        @pl.when(jnp.logical_and(vb < NNZ, vb + CHUNK > NNZ))
        def _partial():
            pltpu.sync_copy(feat.at[pl.ds(vb, NNZ % CHUNK)],
                            fbuf.at[pl.ds(0, NNZ % CHUNK)])

        @pl.when(vb < NNZ)
        def _scatter():
            for hv in range(2):
                pltpu.sync_copy(fbuf.at[pl.ds(hv * (CHUNK // 2), CHUNK // 2)],
                                inter.at[destb.at[hv]])
                pltpu.sync_copy(obuf, valid.at[flagb.at[hv]])


@functools.partial(
    pl.kernel,
    out_type=(
        jax.ShapeDtypeStruct((RINTER, C), jnp.float32),
        jax.ShapeDtypeStruct((NC * RINTER,), jnp.float32),
    ),
    mesh=plsc.VectorSubcoreMesh(core_axis_name="c", subcore_axis_name="s"),
    scratch_types=[
        pltpu.VMEM((CHUNK,), jnp.int32),
        pltpu.VMEM((CHUNK,), jnp.int32),
        pltpu.VMEM((CHUNK,), jnp.int32),
        pltpu.VMEM((CHUNK,), jnp.int32),
        pltpu.VMEM((CHUNK, C), jnp.float32),
        pltpu.VMEM((2, CHUNK // 2), jnp.int32),
        pltpu.VMEM((2, CHUNK // 2), jnp.int32),
        pltpu.VMEM((ZBUF,), jnp.float32),
        pltpu.VMEM((CHUNK // 2,), jnp.float32),
    ],
)
def _sc_scatter(feat, b_hbm, d_hbm, h_hbm, w_hbm, inter, valid, *scratch):
    _sc_body(feat, b_hbm, d_hbm, h_hbm, w_hbm, inter, valid, *scratch)


def _tc_body(xref, vref, oref, xt, sem):
    g = pl.program_id(0)
    x = xref[...]
    v = vref[0, :] + vref[1, :]
    xt[...] = jnp.where(v[None, :] > 0.0, x.T, 0.0)[:, :HW]
    b = g // D
    d = g % D
    cp = pltpu.make_async_copy(xt, oref.at[b, :, d, :], sem)
    cp.start()
    cp.wait()


def _tc_transpose(inter, valid2):
    return pl.pallas_call(
        _tc_body,
        grid=(PLANES,),
        in_specs=[
            pl.BlockSpec((HWP, C), lambda g: (g, 0)),
            pl.BlockSpec((NC, HWP), lambda g: (0, g)),
        ],
        out_specs=pl.BlockSpec(memory_space=pl.ANY),
        out_shape=jax.ShapeDtypeStruct((N_BATCH, C, D, HW), jnp.float32),
        scratch_shapes=[
            pltpu.VMEM((C, HW), jnp.float32),
            pltpu.SemaphoreType.DMA,
        ],
    )(inter, valid2)


def kernel(features, b_idx, d_idx, h_idx, w_idx):
    pad = VPAD - b_idx.shape[0]
    bp = jnp.pad(b_idx, (0, pad))
    dp = jnp.pad(d_idx, (0, pad))
    hp = jnp.pad(h_idx, (0, pad))
    wp = jnp.pad(w_idx, (0, pad))
    inter, valid = _sc_scatter(features, bp, dp, hp, wp)
    out5 = _tc_transpose(inter, valid.reshape(NC, RINTER))
    return out5.reshape(N_BATCH, C * D, H, W)


# speedup vs baseline: 1.1279x; 1.0264x over previous
import functools

import jax
import jax.numpy as jnp
from jax import lax
from jax.experimental import pallas as pl
from jax.experimental.pallas import tpu as pltpu
from jax.experimental.pallas import tpu_sc as plsc

N_BATCH, C, D, H, W = 4, 128, 5, 75, 75
HW = H * W
HWP = 5632
PLANES = N_BATCH * D
RINTER = PLANES * HWP
NNZ = 40000
CHUNK = 256
NCH = 5
NC, NS, L = 2, 16, 16
NW = NC * NS
ZSLICE = (NC * RINTER) // NW
ZBUF = 704
HALF = CHUNK // 2


def _sc_body(feat, b_hbm, d_hbm, h_hbm, w_hbm, inter, valid,
             bbuf, dbuf, hbuf, wbuf, fbuf, destb, flagb, zbuf, obuf,
             zsem, isem, gsem, ssem):
    c = lax.axis_index("c")
    s = lax.axis_index("s")
    wid = s * NC + c

    zv = jnp.zeros((L,), jnp.float32)
    for k in range(ZBUF // L):
        zbuf[pl.ds(k * L, L)] = zv
    ov = jnp.full((L,), 1.0, jnp.float32)
    for k in range(HALF // L):
        obuf[pl.ds(k * L, L)] = ov

    zbase = c * RINTER + s * ZSLICE
    zcps = []
    for t in range(ZSLICE // ZBUF):
        cp = pltpu.make_async_copy(
            zbuf, valid.at[pl.ds(zbase + t * ZBUF, ZBUF)], zsem)
        cp.start()
        zcps.append(cp)

    vbs = [jnp.minimum((wid * NCH + j) * CHUNK, NNZ - CHUNK) for j in range(NCH)]
    icps = []
    for j in range(NCH):
        for src, dst in ((b_hbm, bbuf), (d_hbm, dbuf), (h_hbm, hbuf), (w_hbm, wbuf)):
            cp = pltpu.make_async_copy(
                src.at[pl.ds(vbs[j], CHUNK)], dst.at[pl.ds(j * CHUNK, CHUNK)], isem)
            cp.start()
            icps.append(cp)
    for cp in icps:
        cp.wait()

    for j in range(NCH):
        for v in range(CHUNK // L):
            off = v * L
            bb = bbuf[pl.ds(j * CHUNK + off, L)]
            dd = dbuf[pl.ds(j * CHUNK + off, L)]
            hh = hbuf[pl.ds(j * CHUNK + off, L)]
            ww = wbuf[pl.ds(j * CHUNK + off, L)]
            r = (bb * D + dd) * HWP + hh * W + ww
            hv, lo = off // HALF, off % HALF
            destb[2 * j + hv, pl.ds(lo, L)] = r
            flagb[2 * j + hv, pl.ds(lo, L)] = r + c * RINTER

    for cp in zcps:
        cp.wait()
    plsc.subcore_barrier()

    def start_gather(j):
        cp = pltpu.make_async_copy(
            feat.at[pl.ds(vbs[j], CHUNK)],
            fbuf.at[pl.ds((j % 2) * CHUNK, CHUNK)], gsem)
        cp.start()
        return cp

    def start_scatters(j):
        cps = []
        for hv in range(2):
            cp = pltpu.make_async_copy(
                fbuf.at[pl.ds((j % 2) * CHUNK + hv * HALF, HALF)],
                inter.at[destb.at[2 * j + hv]], ssem)
            cp.start()
            cps.append(cp)
            cp = pltpu.make_async_copy(obuf, valid.at[flagb.at[2 * j + hv]], ssem)
            cp.start()
            cps.append(cp)
        return cps

    gcps, scps = [None] * NCH, [None] * NCH
    for j in range(NCH):
        if j >= 2:
            for cp in scps[j - 2]:
                cp.wait()
        gcps[j] = start_gather(j)
        if j >= 1:
            gcps[j - 1].wait()
            scps[j - 1] = start_scatters(j - 1)
    gcps[NCH - 1].wait()
    scps[NCH - 1] = start_scatters(NCH - 1)
    for j in (NCH - 2, NCH - 1):
        for cp in scps[j]:
            cp.wait()


@functools.partial(
    pl.kernel,
    out_type=(
        jax.ShapeDtypeStruct((RINTER, C), jnp.float32),
        jax.ShapeDtypeStruct((NC * RINTER,), jnp.float32),
    ),
    mesh=plsc.VectorSubcoreMesh(core_axis_name="c", subcore_axis_name="s"),
    scratch_types=[
        pltpu.VMEM((NCH * CHUNK,), jnp.int32),
        pltpu.VMEM((NCH * CHUNK,), jnp.int32),
        pltpu.VMEM((NCH * CHUNK,), jnp.int32),
        pltpu.VMEM((NCH * CHUNK,), jnp.int32),
        pltpu.VMEM((2 * CHUNK, C), jnp.float32),
        pltpu.VMEM((NCH * 2, HALF), jnp.int32),
        pltpu.VMEM((NCH * 2, HALF), jnp.int32),
        pltpu.VMEM((ZBUF,), jnp.float32),
        pltpu.VMEM((HALF,), jnp.float32),
        pltpu.SemaphoreType.DMA,
        pltpu.SemaphoreType.DMA,
        pltpu.SemaphoreType.DMA,
        pltpu.SemaphoreType.DMA,
    ],
)
def _sc_scatter(feat, b_hbm, d_hbm, h_hbm, w_hbm, inter, valid, *scratch):
    _sc_body(feat, b_hbm, d_hbm, h_hbm, w_hbm, inter, valid, *scratch)


def _tc_body(xref, vref, oref):
    d = pl.program_id(1)
    x = xref[...]
    v = vref[0, :] + vref[1, :]
    res = jnp.where(v[None, :] > 0.0, x.T, 0.0)[:, :HW]
    oref[0, :, pl.ds(d, 1), :] = res[:, None, :]


def _tc_transpose(inter, valid2):
    return pl.pallas_call(
        _tc_body,
        grid=(N_BATCH, D),
        in_specs=[
            pl.BlockSpec((HWP, C), lambda b, d: (b * D + d, 0)),
            pl.BlockSpec((NC, HWP), lambda b, d: (0, b * D + d)),
        ],
        out_specs=pl.BlockSpec((1, C, D, HW), lambda b, d: (b, 0, 0, 0)),
        out_shape=jax.ShapeDtypeStruct((N_BATCH, C, D, HW), jnp.float32),
        compiler_params=pltpu.CompilerParams(vmem_limit_bytes=60 * 1024 * 1024),
    )(inter, valid2)


def kernel(features, b_idx, d_idx, h_idx, w_idx):
    inter, valid = _sc_scatter(features, b_idx, d_idx, h_idx, w_idx)
    out5 = _tc_transpose(inter, valid.reshape(NC, RINTER))
    return out5.reshape(N_BATCH, C * D, H, W)
